# baseline (speedup 1.0000x reference)
import jax
import jax.numpy as jnp
from jax import lax
from jax.experimental import pallas as pl
from jax.experimental.pallas import tpu as pltpu
from jax.experimental.pallas import tpu_sc as plsc

N = 10000
E = 160000
D = 256
H = 128
NP = 10240
PE = 163840
NSC = 2
NT = 16
CH = 128

ECH = 32
NBUF = 6
NCH = PE // NT // ECH
SRC_ROWS = PE // NT // CH
SUB = CH // ECH
NSUP = NCH // 16

_MESH = dict(core_axis_name="c", subcore_axis_name="s")


def _hist_body(srch, out_hbm, accum, idx_v, ones_v, zbuf):
    c = lax.axis_index("c")
    s = lax.axis_index("s")
    t = c * NT + s

    pltpu.sync_copy(srch.at[pl.ds(40 * t, 40)], idx_v)

    for k in range(8):
        ones_v[pl.ds(16 * k, 16)] = jnp.ones((16,), jnp.float32)

    def _zb(i, _):
        zbuf[pl.ds(pl.multiple_of(16 * i, 16), 16)] = jnp.zeros((16,), jnp.float32)
        return 0

    lax.fori_loop(0, 40, _zb, 0)

    pltpu.sync_copy(zbuf, accum.at[pl.ds(640 * s, 640)])
    plsc.subcore_barrier()

    def _body(j, _):
        pltpu.sync_copy(ones_v, accum.at[idx_v.at[j]], add=True)
        return 0

    lax.fori_loop(0, 40, _body, 0)
    plsc.subcore_barrier()

    pltpu.sync_copy(accum.at[pl.ds(640 * s, 640)], out_hbm.at[c].at[pl.ds(640 * s, 640)])


_hist = pl.kernel(
    _hist_body,
    out_type=jax.ShapeDtypeStruct((NSC, NP), jnp.float32),
    mesh=plsc.VectorSubcoreMesh(**_MESH),
    scratch_types=[
        pltpu.VMEM_SHARED((NP,), jnp.float32),
        pltpu.VMEM((40, CH), jnp.int32),
        pltpu.VMEM((CH,), jnp.float32),
        pltpu.VMEM((640,), jnp.float32),
    ],
)


def _agg_body(srcagg, dstp, y_hbm, out_hbm, accum, src_v, dst_v, rows_v, zbuf,
              sem_g, sem_s, sem_i):
    c = lax.axis_index("c")
    s = lax.axis_index("s")

    pltpu.sync_copy(srcagg.at[c].at[pl.ds(SRC_ROWS * s, SRC_ROWS)], src_v)
    pltpu.sync_copy(dstp.at[pl.ds(NSUP * s, 2)], dst_v)

    def _zb(i, _):
        r = i // 8
        k = i % 8
        zbuf[r, pl.ds(pl.multiple_of(16 * k, 16), 16)] = jnp.zeros((16,), jnp.float32)
        return 0

    lax.fori_loop(0, 64 * 8, _zb, 0)
    for q in range(10):
        pltpu.sync_copy(zbuf, accum.at[pl.ds(640 * s + 64 * q, 64)])
    plsc.subcore_barrier()

    def _g(j, b):
        idx = src_v.at[j // SUB].at[pl.ds(pl.multiple_of(ECH * (j % SUB), ECH), ECH)]
        return pltpu.make_async_copy(y_hbm.at[idx], rows_v.at[b], sem_g.at[b])

    def _s(j, b):
        slot = lax.rem(j // 16, 2)
        idx = dst_v.at[slot].at[lax.rem(j, 16)]
        return pltpu.make_async_copy(rows_v.at[b], accum.at[idx], sem_s.at[b])

    def _i(u):
        slot = lax.rem(u, 2)
        return pltpu.make_async_copy(dstp.at[NSUP * s + u], dst_v.at[slot],
                                     sem_i.at[slot])

    for b0 in range(NBUF):
        _g(b0, b0).start()

    def _body(j, _):
        b = lax.rem(j, NBUF)
        _g(j, b).wait()

        @pl.when((lax.rem(j, 16) == 0) & (j >= 32))
        def _():
            _i(j // 16).wait()

        _s(j, b).start(add=True)

        @pl.when(j >= 3)
        def _():
            b1 = lax.rem(j + NBUF - 3, NBUF)
            _s(j - 3, b1).wait()

            @pl.when(j + NBUF - 3 < NCH)
            def _():
                _g(j + NBUF - 3, b1).start()

        @pl.when((lax.rem(j, 16) == 2) & (j >= 18) & (j // 16 + 1 < NSUP))
        def _():
            _i(j // 16 + 1).start()

        return 0

    lax.fori_loop(0, NCH, _body, 0)
    for dj in range(3):
        _s(NCH - 3 + dj, lax.rem(NCH - 3 + dj, NBUF)).wait()
    plsc.subcore_barrier()

    pltpu.sync_copy(accum.at[pl.ds(640 * s, 640)], out_hbm.at[c].at[pl.ds(640 * s, 640)])


_agg = pl.kernel(
    _agg_body,
    out_type=jax.ShapeDtypeStruct((NSC, NP, H), jnp.float32),
    mesh=plsc.VectorSubcoreMesh(**_MESH),
    scratch_types=[
        pltpu.VMEM_SHARED((NP, H), jnp.float32),
        pltpu.VMEM((SRC_ROWS, CH), jnp.int32),
        pltpu.VMEM((2, 16, ECH), jnp.int32),
        pltpu.VMEM((NBUF, ECH, H), jnp.float32),
        pltpu.VMEM((64, H), jnp.float32),
        pltpu.SemaphoreType.DMA((NBUF,)),
        pltpu.SemaphoreType.DMA((NBUF,)),
        pltpu.SemaphoreType.DMA((2,)),
    ],
)


def _mm_body(x_ref, w_ref, deg_ref, y_ref):
    rdeg = lax.rsqrt(deg_ref[0] + deg_ref[1])
    z = lax.dot_general(
        x_ref[...], w_ref[...],
        (((1,), (1,)), ((), ())),
        preferred_element_type=jnp.float32,
        precision=lax.Precision.HIGHEST,
    )
    y_ref[...] = z * rdeg


def _tc_mm(x, W, deg3):
    R = 1000
    return pl.pallas_call(
        _mm_body,
        grid=(N // R, NSC),
        in_specs=[
            pl.BlockSpec((R, D), lambda i, h: (i, 0)),
            pl.BlockSpec((H, D), lambda i, h: (h, 0)),
            pl.BlockSpec((NSC, R, 1), lambda i, h: (0, i, 0)),
        ],
        out_specs=pl.BlockSpec((R, H), lambda i, h: (h * (N // R) + i, 0)),
        out_shape=jax.ShapeDtypeStruct((NSC * N, H), jnp.float32),
    )(x, W, deg3)


def _ep_body(agg_ref, deg_ref, b_ref, out_ref):
    rdeg = lax.rsqrt(deg_ref[0] + deg_ref[1])
    out_ref[:, :H] = agg_ref[0] * rdeg + b_ref[0, :H]
    out_ref[:, H:] = agg_ref[1] * rdeg + b_ref[0, H:]


def _tc_ep(agg, deg3, b2):
    R = 1000
    return pl.pallas_call(
        _ep_body,
        grid=(N // R,),
        in_specs=[
            pl.BlockSpec((NSC, R, H), lambda i: (0, i, 0)),
            pl.BlockSpec((NSC, R, 1), lambda i: (0, i, 0)),
            pl.BlockSpec((1, D), lambda i: (0, 0)),
        ],
        out_specs=pl.BlockSpec((R, D), lambda i: (i, 0)),
        out_shape=jax.ShapeDtypeStruct((N, D), jnp.float32),
    )(agg, deg3, b2)


def kernel(x, edge_index, W, b):
    src = edge_index[0]
    dst = edge_index[1]
    pad = PE - E
    dummy = N + (jnp.arange(pad, dtype=jnp.int32) % (NP - N))
    srch = jnp.concatenate([src, dummy]).reshape(PE // CH, CH)
    src0 = jnp.concatenate([src, jnp.zeros((pad,), jnp.int32)])
    srcagg = jnp.stack([src0, src0 + N]).reshape(NSC, NT * SRC_ROWS, CH)
    dstp = jnp.concatenate([dst, dummy]).reshape(NT * NSUP, 16, ECH)

    deg2 = _hist(srch)
    deg3 = deg2.reshape(NSC, NP, 1)
    y = _tc_mm(x, W, deg3)
    agg = _agg(srcagg, dstp, y)
    return _tc_ep(agg, deg3, b.reshape(1, D))


# speedup vs baseline: 1.0565x; 1.0565x over previous
import jax
import jax.numpy as jnp
from jax import lax
from jax.experimental import pallas as pl
from jax.experimental.pallas import tpu as pltpu
from jax.experimental.pallas import tpu_sc as plsc

N = 10000
E = 160000
D = 256
H = 128
NP = 10240
PE = 163840
NSC = 2
NT = 16
CH = 128

ECH = 64
NBUF = 3
NCH = PE // NT // ECH
SRC_ROWS = PE // NT // CH
SUB = CH // ECH
NSUP = NCH // 16

_MESH = dict(core_axis_name="c", subcore_axis_name="s")


def _hist_body(srch, out_hbm, accum, idx_v, ones_v, zbuf):
    c = lax.axis_index("c")
    s = lax.axis_index("s")
    t = c * NT + s

    pltpu.sync_copy(srch.at[pl.ds(40 * t, 40)], idx_v)

    for k in range(8):
        ones_v[pl.ds(16 * k, 16)] = jnp.ones((16,), jnp.float32)

    def _zb(i, _):
        zbuf[pl.ds(pl.multiple_of(16 * i, 16), 16)] = jnp.zeros((16,), jnp.float32)
        return 0

    lax.fori_loop(0, 40, _zb, 0)

    pltpu.sync_copy(zbuf, accum.at[pl.ds(640 * s, 640)])
    plsc.subcore_barrier()

    def _body(j, _):
        pltpu.sync_copy(ones_v, accum.at[idx_v.at[j]], add=True)
        return 0

    lax.fori_loop(0, 40, _body, 0)
    plsc.subcore_barrier()

    pltpu.sync_copy(accum.at[pl.ds(640 * s, 640)], out_hbm.at[c].at[pl.ds(640 * s, 640)])


_hist = pl.kernel(
    _hist_body,
    out_type=jax.ShapeDtypeStruct((NSC, NP), jnp.float32),
    mesh=plsc.VectorSubcoreMesh(**_MESH),
    scratch_types=[
        pltpu.VMEM_SHARED((NP,), jnp.float32),
        pltpu.VMEM((40, CH), jnp.int32),
        pltpu.VMEM((CH,), jnp.float32),
        pltpu.VMEM((640,), jnp.float32),
    ],
)


def _agg_body(srcagg, dstp, y_hbm, out_hbm, accum, src_v, dst_v, rows_v, zbuf,
              sem_g, sem_s, sem_i):
    c = lax.axis_index("c")
    s = lax.axis_index("s")

    pltpu.sync_copy(srcagg.at[c].at[pl.ds(SRC_ROWS * s, SRC_ROWS)], src_v)
    pltpu.sync_copy(dstp.at[pl.ds(NSUP * s, 2)], dst_v)

    def _zb(i, _):
        r = i // 8
        k = i % 8
        zbuf[r, pl.ds(pl.multiple_of(16 * k, 16), 16)] = jnp.zeros((16,), jnp.float32)
        return 0

    lax.fori_loop(0, 64 * 8, _zb, 0)
    for q in range(10):
        pltpu.sync_copy(zbuf, accum.at[pl.ds(640 * s + 64 * q, 64)])
    plsc.subcore_barrier()

    def _g(j, b):
        idx = src_v.at[j // SUB].at[pl.ds(pl.multiple_of(ECH * (j % SUB), ECH), ECH)]
        return pltpu.make_async_copy(y_hbm.at[idx], rows_v.at[b], sem_g.at[b])

    def _s(j, b):
        slot = lax.rem(j // 16, 2)
        idx = dst_v.at[slot].at[lax.rem(j, 16)]
        return pltpu.make_async_copy(rows_v.at[b], accum.at[idx], sem_s.at[b])

    def _i(u):
        slot = lax.rem(u, 2)
        return pltpu.make_async_copy(dstp.at[NSUP * s + u], dst_v.at[slot],
                                     sem_i.at[slot])

    for b0 in range(NBUF):
        _g(b0, b0).start()

    def _body(j, _):
        b = lax.rem(j, NBUF)
        _g(j, b).wait()

        @pl.when((lax.rem(j, 16) == 0) & (j >= 32))
        def _():
            _i(j // 16).wait()

        _s(j, b).start(add=True)

        @pl.when(j >= 1)
        def _():
            b1 = lax.rem(j + NBUF - 1, NBUF)
            _s(j - 1, b1).wait()

            @pl.when(j + NBUF - 1 < NCH)
            def _():
                _g(j + NBUF - 1, b1).start()

        @pl.when((lax.rem(j, 16) == 0) & (j >= 16) & (j // 16 + 1 < NSUP))
        def _():
            _i(j // 16 + 1).start()

        return 0

    lax.fori_loop(0, NCH, _body, 0)
    _s(NCH - 1, lax.rem(NCH - 1, NBUF)).wait()
    plsc.subcore_barrier()

    pltpu.sync_copy(accum.at[pl.ds(640 * s, 640)], out_hbm.at[c].at[pl.ds(640 * s, 640)])


_agg = pl.kernel(
    _agg_body,
    out_type=jax.ShapeDtypeStruct((NSC, NP, H), jnp.float32),
    mesh=plsc.VectorSubcoreMesh(**_MESH),
    scratch_types=[
        pltpu.VMEM_SHARED((NP, H), jnp.float32),
        pltpu.VMEM((SRC_ROWS, CH), jnp.int32),
        pltpu.VMEM((2, 16, ECH), jnp.int32),
        pltpu.VMEM((NBUF, ECH, H), jnp.float32),
        pltpu.VMEM((64, H), jnp.float32),
        pltpu.SemaphoreType.DMA((NBUF,)),
        pltpu.SemaphoreType.DMA((NBUF,)),
        pltpu.SemaphoreType.DMA((2,)),
    ],
)


def _mm_body(x_ref, w_ref, deg_ref, y_ref):
    rdeg = lax.rsqrt(deg_ref[0] + deg_ref[1])
    z = lax.dot_general(
        x_ref[...], w_ref[...],
        (((1,), (1,)), ((), ())),
        preferred_element_type=jnp.float32,
        precision=lax.Precision.HIGHEST,
    )
    y_ref[...] = z * rdeg


def _tc_mm(x, W, deg3):
    R = 1000
    return pl.pallas_call(
        _mm_body,
        grid=(N // R, NSC),
        in_specs=[
            pl.BlockSpec((R, D), lambda i, h: (i, 0)),
            pl.BlockSpec((H, D), lambda i, h: (h, 0)),
            pl.BlockSpec((NSC, R, 1), lambda i, h: (0, i, 0)),
        ],
        out_specs=pl.BlockSpec((R, H), lambda i, h: (h * (N // R) + i, 0)),
        out_shape=jax.ShapeDtypeStruct((NSC * N, H), jnp.float32),
    )(x, W, deg3)


def _ep_body(agg_ref, deg_ref, b_ref, out_ref):
    rdeg = lax.rsqrt(deg_ref[0] + deg_ref[1])
    out_ref[:, :H] = agg_ref[0] * rdeg + b_ref[0, :H]
    out_ref[:, H:] = agg_ref[1] * rdeg + b_ref[0, H:]


def _tc_ep(agg, deg3, b2):
    R = 1000
    return pl.pallas_call(
        _ep_body,
        grid=(N // R,),
        in_specs=[
            pl.BlockSpec((NSC, R, H), lambda i: (0, i, 0)),
            pl.BlockSpec((NSC, R, 1), lambda i: (0, i, 0)),
            pl.BlockSpec((1, D), lambda i: (0, 0)),
        ],
        out_specs=pl.BlockSpec((R, D), lambda i: (i, 0)),
        out_shape=jax.ShapeDtypeStruct((N, D), jnp.float32),
    )(agg, deg3, b2)


def kernel(x, edge_index, W, b):
    src = edge_index[0]
    dst = edge_index[1]
    pad = PE - E
    dummy = N + (jnp.arange(pad, dtype=jnp.int32) % (NP - N))
    srch = jnp.concatenate([src, dummy]).reshape(PE // CH, CH)
    src0 = jnp.concatenate([src, jnp.zeros((pad,), jnp.int32)])
    srcagg = jnp.stack([src0, src0 + N]).reshape(NSC, NT * SRC_ROWS, CH)
    dstp = jnp.concatenate([dst, dummy]).reshape(NT * NSUP, 16, ECH)

    deg2 = _hist(srch)
    deg3 = deg2.reshape(NSC, NP, 1)
    y = _tc_mm(x, W, deg3)
    agg = _agg(srcagg, dstp, y)
    return _tc_ep(agg, deg3, b.reshape(1, D))
